# baseline (speedup 1.0000x reference)
import jax
import jax.numpy as jnp
from jax.experimental import pallas as pl
from jax.experimental.pallas import tpu as pltpu


def _pick_tile(dim, candidates):
    for t in candidates:
        if dim % t == 0:
            return t
    return dim


def _fused_body(adj_a_ref, adj_b_ref, x_ref, w_ref, b_ref, o_ref):
    xb = x_ref[...].astype(jnp.bfloat16)
    wb = w_ref[...].astype(jnp.bfloat16)
    half = adj_a_ref.shape[0]
    for idx, a_ref in enumerate((adj_a_ref, adj_b_ref)):
        t = jnp.dot(
            a_ref[...].astype(jnp.bfloat16), xb,
            preferred_element_type=jnp.float32,
        )
        o_ref[idx * half:(idx + 1) * half, :] = (
            jnp.dot(t.astype(jnp.bfloat16), wb,
                    preferred_element_type=jnp.float32)
            + b_ref[...]
        ).astype(o_ref.dtype)


def kernel(x, adj, weight, bias):
    n, fin = x.shape
    fout = weight.shape[1]
    tm = _pick_tile(n, (512, 256, 128))
    grid = (n // tm,)

    bias2d = bias.astype(jnp.float32).reshape(1, fout)
    cost = pl.CostEstimate(
        flops=2 * n * n * fin + 2 * n * fin * fout,
        transcendentals=0,
        bytes_accessed=4 * (n * n + n * fin + fin * fout + fout + n * fout),
    )
    return pl.pallas_call(
        _fused_body,
        out_shape=jax.ShapeDtypeStruct((n, fout), jnp.float32),
        grid_spec=pltpu.PrefetchScalarGridSpec(
            num_scalar_prefetch=0,
            grid=grid,
            in_specs=[
                pl.BlockSpec((tm // 2, n), lambda i: (2 * i, 0)),
                pl.BlockSpec((tm // 2, n), lambda i: (2 * i + 1, 0)),
                pl.BlockSpec((n, fin), lambda i: (0, 0)),
                pl.BlockSpec((fin, fout), lambda i: (0, 0)),
                pl.BlockSpec((1, fout), lambda i: (0, 0)),
            ],
            out_specs=pl.BlockSpec((tm, fout), lambda i: (i, 0)),
        ),
        compiler_params=pltpu.CompilerParams(
            dimension_semantics=("parallel",),
            vmem_limit_bytes=100 * 1024 * 1024,
        ),
        cost_estimate=cost,
    )(adj, adj, x.astype(jnp.float32), weight.astype(jnp.float32), bias2d)


# speedup vs baseline: 1.1519x; 1.1519x over previous
import jax
import jax.numpy as jnp
from jax.experimental import pallas as pl
from jax.experimental.pallas import tpu as pltpu


def _pick_tile(dim, candidates):
    for t in candidates:
        if dim % t == 0:
            return t
    return dim


def _fused_body(adj_ref, x_ref, w_ref, b_ref, o_ref):
    t = jnp.dot(
        adj_ref[...].astype(jnp.bfloat16),
        x_ref[...].astype(jnp.bfloat16),
        preferred_element_type=jnp.float32,
    )
    o_ref[...] = (
        jnp.dot(
            t.astype(jnp.bfloat16),
            w_ref[...].astype(jnp.bfloat16),
            preferred_element_type=jnp.float32,
        )
        + b_ref[...]
    ).astype(o_ref.dtype)


def kernel(x, adj, weight, bias):
    n, fin = x.shape
    fout = weight.shape[1]
    tm = _pick_tile(n, (512, 256, 128))
    grid = (n // tm,)

    bias2d = bias.astype(jnp.float32).reshape(1, fout)
    cost = pl.CostEstimate(
        flops=2 * n * n * fin + 2 * n * fin * fout,
        transcendentals=0,
        bytes_accessed=4 * (n * n + n * fin + fin * fout + fout + n * fout),
    )
    return pl.pallas_call(
        _fused_body,
        out_shape=jax.ShapeDtypeStruct((n, fout), jnp.float32),
        grid_spec=pltpu.PrefetchScalarGridSpec(
            num_scalar_prefetch=0,
            grid=grid,
            in_specs=[
                pl.BlockSpec((tm, n), lambda i: (i, 0)),
                pl.BlockSpec((n, fin), lambda i: (0, 0)),
                pl.BlockSpec((fin, fout), lambda i: (0, 0)),
                pl.BlockSpec((1, fout), lambda i: (0, 0)),
            ],
            out_specs=pl.BlockSpec((tm, fout), lambda i: (i, 0)),
        ),
        compiler_params=pltpu.CompilerParams(
            dimension_semantics=("parallel",),
            vmem_limit_bytes=100 * 1024 * 1024,
        ),
        cost_estimate=cost,
    )(adj, x.astype(jnp.float32), weight.astype(jnp.float32), bias2d)
